# Initial kernel scaffold; baseline (speedup 1.0000x reference)
import jax
import jax.numpy as jnp
from jax.experimental import pallas as pl


def kernel(x, edge_index, edge_weight, adj_w1, adj_w2, conv1_w, conv1_b, conv2_w, conv2_b):
    raise NotImplementedError("write your pallas kernel here")


# speedup vs baseline: 1.1496x; 1.1496x over previous
import functools

import jax
import jax.numpy as jnp
from jax.experimental import pallas as pl
from jax.experimental.pallas import tpu as pltpu

NUM_NODES = 25200
NUM_EDGES = 504000
ROW_BLK = 1008


def _mlp_kernel(ew_ref, w1_ref, w2_ref, out_ref):
    t = ew_ref[...].reshape(1, -1)
    t = t @ w1_ref[...].T
    t = jnp.where(t > 0, t, jnp.exp(t) - 1.0)
    t = t @ w2_ref[...].T
    t = jnp.tanh(t)
    t = jnp.maximum(t, 0.0)
    out_ref[...] = t.reshape(-1, 1)


def _edge_mlp(edge_weight, adj_w1, adj_w2):
    return pl.pallas_call(
        _mlp_kernel,
        out_shape=jax.ShapeDtypeStruct((420, 1), jnp.float32),
    )(edge_weight, adj_w1, adj_w2)


def _conv1_mm_kernel(t0, t1, t2, t3, t4, w, b, out_ref):
    acc = jnp.dot(t0[...], w[0], preferred_element_type=jnp.float32)
    acc += jnp.dot(t1[...], w[1], preferred_element_type=jnp.float32)
    acc += jnp.dot(t2[...], w[2], preferred_element_type=jnp.float32)
    acc += jnp.dot(t3[...], w[3], preferred_element_type=jnp.float32)
    acc += jnp.dot(t4[...], w[4], preferred_element_type=jnp.float32)
    out_ref[...] = jnp.maximum(acc + b[...], 0.0)


def _conv1_mm(ts, w, b):
    in_spec = pl.BlockSpec((ROW_BLK, 128), lambda i: (i, 0))
    return pl.pallas_call(
        _conv1_mm_kernel,
        grid=(NUM_NODES // ROW_BLK,),
        in_specs=[in_spec] * 5 + [
            pl.BlockSpec((5, 128, 256), lambda i: (0, 0, 0)),
            pl.BlockSpec((1, 256), lambda i: (0, 0)),
        ],
        out_specs=pl.BlockSpec((ROW_BLK, 256), lambda i: (i, 0)),
        out_shape=jax.ShapeDtypeStruct((NUM_NODES, 256), jnp.float32),
    )(*ts, w, b.reshape(1, 256))


def _proj_mm_kernel(h, w, out_ref):
    out_ref[...] = jnp.dot(h[...], w[...], preferred_element_type=jnp.float32)


def _proj_mm(h, wcat):
    return pl.pallas_call(
        _proj_mm_kernel,
        grid=(NUM_NODES // ROW_BLK,),
        in_specs=[
            pl.BlockSpec((ROW_BLK, 256), lambda i: (i, 0)),
            pl.BlockSpec((256, 320), lambda i: (0, 0)),
        ],
        out_specs=pl.BlockSpec((ROW_BLK, 320), lambda i: (i, 0)),
        out_shape=jax.ShapeDtypeStruct((NUM_NODES, 320), jnp.float32),
    )(h, wcat)


def _scatter(v, col, row, c):
    return jnp.zeros_like(v).at[col].add(c[:, None] * v[row])


def kernel(x, edge_index, edge_weight, adj_w1, adj_w2, conv1_w, conv1_b, conv2_w, conv2_b):
    ew = _edge_mlp(edge_weight, adj_w1, adj_w2)
    reps = edge_index.shape[-1] // 420
    train_ew = jnp.tile(ew, (reps, 1))

    row, col = edge_index[0], edge_index[1]
    w = train_ew.reshape(-1)
    deg = jnp.zeros((NUM_NODES,), jnp.float32).at[row].add(w)
    dis = jnp.where(deg > 0, jax.lax.rsqrt(deg), 0.0)
    norm = -dis[row] * w * dis[col]
    norm2 = 2.0 * norm

    t0 = x
    t1 = _scatter(t0, col, row, norm)
    t2 = _scatter(t1, col, row, norm2) - t0
    t3 = _scatter(t2, col, row, norm2) - t1
    t4 = _scatter(t3, col, row, norm2) - t2
    h = _conv1_mm([t0, t1, t2, t3, t4], conv1_w, conv1_b)

    w2cat = jnp.concatenate([conv2_w[k] for k in range(5)], axis=1)
    yy = _proj_mm(h, w2cat)
    y = [yy[:, 64 * k:64 * k + 64] for k in range(5)]
    b4 = y[4]
    b3 = _scatter(b4, col, row, norm2) + y[3]
    b2 = _scatter(b3, col, row, norm2) + y[2] - b4
    b1 = _scatter(b2, col, row, norm2) + y[1] - b3
    out = _scatter(b1, col, row, norm) + y[0] - b2 + conv2_b

    return (out, ew, train_ew)


# speedup vs baseline: 3.0486x; 2.6518x over previous
import functools

import jax
import jax.numpy as jnp
from jax import lax
from jax.experimental import pallas as pl
from jax.experimental.pallas import tpu as pltpu
from jax.experimental.pallas import tpu_sc as plsc

N = 25200
E = 504000
N_PAD = 25600
E_PAD = 524288
EPT = E_PAD // 32
NCH = EPT // 128
RPT = N_PAD // 16
GRP = 32
ROW_BLK = 1008

@functools.cache
def _mesh():
    return plsc.VectorSubcoreMesh(
        core_axis_name="c", subcore_axis_name="s", num_cores=2, num_subcores=16)


def _mlp_kernel(ew_ref, w1_ref, w2_ref, out_ref):
    t = ew_ref[...].reshape(1, -1)
    t = t @ w1_ref[...].T
    t = jnp.where(t > 0, t, jnp.exp(t) - 1.0)
    t = t @ w2_ref[...].T
    t = jnp.tanh(t)
    t = jnp.maximum(t, 0.0)
    out_ref[...] = t.reshape(-1, 1)


def _edge_mlp(edge_weight, adj_w1, adj_w2):
    return pl.pallas_call(
        _mlp_kernel,
        out_shape=jax.ShapeDtypeStruct((420, 1), jnp.float32),
    )(edge_weight, adj_w1, adj_w2)


def _dis_kernel(degs_ref, out_ref):
    deg = jnp.sum(degs_ref[...], axis=0)
    out_ref[...] = jnp.where(deg > 0, lax.rsqrt(deg), 0.0)


def _dis(deg_parts):
    return pl.pallas_call(
        _dis_kernel,
        out_shape=jax.ShapeDtypeStruct((200, 128), jnp.float32),
    )(deg_parts)


def _conv1_mm_kernel(t0, t1, t2, t3, t4, w, b, out_ref):
    acc = jnp.dot(t0[...], w[0], preferred_element_type=jnp.float32)
    acc += jnp.dot(t1[...], w[1], preferred_element_type=jnp.float32)
    acc += jnp.dot(t2[...], w[2], preferred_element_type=jnp.float32)
    acc += jnp.dot(t3[...], w[3], preferred_element_type=jnp.float32)
    acc += jnp.dot(t4[...], w[4], preferred_element_type=jnp.float32)
    out_ref[...] = jnp.maximum(acc + b[...], 0.0)


def _conv1_mm(ts, w, b):
    in_spec = pl.BlockSpec((ROW_BLK, 128), lambda i: (i, 0))
    return pl.pallas_call(
        _conv1_mm_kernel,
        grid=(N // ROW_BLK,),
        in_specs=[in_spec] * 5 + [
            pl.BlockSpec((5, 128, 256), lambda i: (0, 0, 0)),
            pl.BlockSpec((1, 256), lambda i: (0, 0)),
        ],
        out_specs=pl.BlockSpec((ROW_BLK, 256), lambda i: (i, 0)),
        out_shape=jax.ShapeDtypeStruct((N, 256), jnp.float32),
    )(*ts, w, b.reshape(1, 256))


def _proj_mm_kernel(h, w, out_ref):
    out_ref[0] = jnp.dot(h[...], w[0], preferred_element_type=jnp.float32)


def _proj_mm(h, w):
    return pl.pallas_call(
        _proj_mm_kernel,
        grid=(5, N // ROW_BLK),
        in_specs=[
            pl.BlockSpec((ROW_BLK, 256), lambda k, i: (i, 0)),
            pl.BlockSpec((1, 256, 64), lambda k, i: (k, 0, 0)),
        ],
        out_specs=pl.BlockSpec((1, ROW_BLK, 64), lambda k, i: (k, i, 0)),
        out_shape=jax.ShapeDtypeStruct((5, N, 64), jnp.float32),
    )(h, w)


def _asm128_kernel(*refs, has_sub):
    if has_sub:
        pa, pb, sub, out_ref = refs
    else:
        pa, pb, out_ref = refs
    left = pa[0] + pa[1]
    right = pb[0] + pb[1]
    res = jnp.concatenate([left, right], axis=1)
    if has_sub:
        res = res - sub[...]
    out_ref[...] = res


def _asm128(pa, pb, sub=None):
    has_sub = sub is not None
    part_spec = pl.BlockSpec((2, ROW_BLK, 64), lambda i: (0, i, 0))
    in_specs = [part_spec, part_spec]
    args = [pa, pb]
    if has_sub:
        in_specs.append(pl.BlockSpec((ROW_BLK, 128), lambda i: (i, 0)))
        args.append(sub)
    return pl.pallas_call(
        functools.partial(_asm128_kernel, has_sub=has_sub),
        grid=(N // ROW_BLK,),
        in_specs=in_specs,
        out_specs=pl.BlockSpec((ROW_BLK, 128), lambda i: (i, 0)),
        out_shape=jax.ShapeDtypeStruct((N, 128), jnp.float32),
    )(*args)


def _comb64_kernel(*refs, has_sub):
    if has_sub:
        p, yy, sub, out_ref = refs
    else:
        p, yy, out_ref = refs
    res = p[0] + p[1] + yy[0]
    if has_sub:
        res = res - sub[...]
    out_ref[...] = res


def _comb64(p, yy, ycol, sub=None):
    has_sub = sub is not None
    in_specs = [
        pl.BlockSpec((2, ROW_BLK, 64), lambda i: (0, i, 0)),
        pl.BlockSpec((1, ROW_BLK, 64), lambda i, c=ycol: (c, i, 0)),
    ]
    args = [p, yy]
    if has_sub:
        in_specs.append(pl.BlockSpec((ROW_BLK, 64), lambda i: (i, 0)))
        args.append(sub)
    return pl.pallas_call(
        functools.partial(_comb64_kernel, has_sub=has_sub),
        grid=(N // ROW_BLK,),
        in_specs=in_specs,
        out_specs=pl.BlockSpec((ROW_BLK, 64), lambda i: (i, 0)),
        out_shape=jax.ShapeDtypeStruct((N, 64), jnp.float32),
    )(*args)


def _wid():
    return lax.axis_index("c") * 16 + lax.axis_index("s")


def _deg_body(rowp, coefp, out, dpriv, idxb, cb):
    w = _wid()
    def zero(i, _):
        dpriv[pl.ds(16 * i, 16)] = jnp.zeros((16,), jnp.float32)
        return 0
    lax.fori_loop(0, N_PAD // 16, zero, 0)
    base = w * EPT
    def chunk(k, _):
        e0 = base + k * 128
        pltpu.sync_copy(rowp.at[pl.ds(e0, 128)], idxb)
        pltpu.sync_copy(coefp.at[pl.ds(e0, 128)], cb)
        def grp(g, _):
            r16 = idxb[pl.ds(16 * g, 16)]
            c16 = cb[pl.ds(16 * g, 16)]
            plsc.addupdate_scatter(dpriv, [r16], c16)
            return 0
        lax.fori_loop(0, 8, grp, 0)
        return 0
    lax.fori_loop(0, NCH, chunk, 0)
    pltpu.sync_copy(dpriv, out.at[pl.ds(w * N_PAD, N_PAD)])


@functools.cache
def _deg_call():
    return pl.kernel(
        _deg_body,
        out_type=jax.ShapeDtypeStruct((32 * N_PAD,), jnp.float32),
        mesh=_mesh(),
        compiler_params=pltpu.CompilerParams(needs_layout_passes=False, use_tc_tiling_on_sc=False),
        scratch_types=[
            pltpu.VMEM((N_PAD,), jnp.float32),
            pltpu.VMEM((128,), jnp.int32),
            pltpu.VMEM((128,), jnp.float32),
        ],
    )


def _norm_body(dis, rowp, colp, coefp, na_out, nb_out, disv, idxr, idxc, cb, na, nb):
    w = _wid()
    pltpu.sync_copy(dis, disv)
    base = w * EPT
    def chunk(k, _):
        e0 = base + k * 128
        pltpu.sync_copy(rowp.at[pl.ds(e0, 128)], idxr)
        pltpu.sync_copy(colp.at[pl.ds(e0, 128)], idxc)
        pltpu.sync_copy(coefp.at[pl.ds(e0, 128)], cb)
        def grp(g, _):
            r16 = idxr[pl.ds(16 * g, 16)]
            c16 = idxc[pl.ds(16 * g, 16)]
            w16 = cb[pl.ds(16 * g, 16)]
            dr = plsc.load_gather(disv, [r16])
            dc = plsc.load_gather(disv, [c16])
            v = -(dr * w16 * dc)
            na[pl.ds(16 * g, 16)] = v
            nb[pl.ds(16 * g, 16)] = v + v
            return 0
        lax.fori_loop(0, 8, grp, 0)
        pltpu.sync_copy(na, na_out.at[pl.ds(e0, 128)])
        pltpu.sync_copy(nb, nb_out.at[pl.ds(e0, 128)])
        return 0
    lax.fori_loop(0, NCH, chunk, 0)


@functools.cache
def _norm_call():
    return pl.kernel(
        _norm_body,
        out_type=(jax.ShapeDtypeStruct((E_PAD,), jnp.float32),
                  jax.ShapeDtypeStruct((E_PAD,), jnp.float32)),
        mesh=_mesh(),
        compiler_params=pltpu.CompilerParams(needs_layout_passes=False, use_tc_tiling_on_sc=False),
        scratch_types=[
            pltpu.VMEM((N_PAD,), jnp.float32),
            pltpu.VMEM((128,), jnp.int32),
            pltpu.VMEM((128,), jnp.int32),
            pltpu.VMEM((128,), jnp.float32),
            pltpu.VMEM((128,), jnp.float32),
            pltpu.VMEM((128,), jnp.float32),
        ],
    )


def _lx_body(v2, rowp2d, colp2d, coefp, out,
             rowb, colb, cfb, gbuf, acc, sem, *, mult, off):
    c = lax.axis_index("c")
    s = lax.axis_index("s")
    w = c * 16 + s
    def zrow(i, _):
        for j in range(4):
            gbuf[i, pl.ds(16 * j, 16)] = jnp.zeros((16,), jnp.float32)
        return 0
    lax.fori_loop(0, 128, zrow, 0)
    def zacc(i, _):
        pltpu.sync_copy(gbuf.at[pl.ds(0, 128), :],
                        acc.at[pl.ds(s * RPT + i * 128, 128), :])
        return 0
    lax.fori_loop(0, RPT // 128, zacc, 0)
    pltpu.sync_copy(gbuf.at[pl.ds(0, RPT % 128), :],
                    acc.at[pl.ds(s * RPT + RPT - RPT % 128, RPT % 128), :])
    plsc.subcore_barrier()
    def group(gi, _):
        pltpu.sync_copy(rowp2d.at[pl.ds(w * NCH + gi * GRP, GRP), :], rowb)
        pltpu.sync_copy(colp2d.at[pl.ds(w * NCH + gi * GRP, GRP), :], colb)
        pltpu.sync_copy(coefp.at[pl.ds(w * EPT + gi * (GRP * 128), GRP * 128)], cfb)
        if mult != 1 or off != 0:
            def xform(k, _):
                def xg(g, _):
                    r16 = rowb[k, pl.ds(16 * g, 16)]
                    rowb[k, pl.ds(16 * g, 16)] = r16 * mult + off
                    return 0
                lax.fori_loop(0, 8, xg, 0)
                return 0
            lax.fori_loop(0, GRP, xform, 0)
        def chunk(k, _):
            pltpu.async_copy(v2.at[rowb.at[k]], gbuf, sem).wait()
            def scale(g, _):
                for l in range(16):
                    i = 16 * g + l
                    cv = plsc.load_gather(cfb, [jnp.full((16,), k * 128 + i, jnp.int32)])
                    for j in range(4):
                        gbuf[i, pl.ds(16 * j, 16)] = gbuf[i, pl.ds(16 * j, 16)] * cv
                return 0
            lax.fori_loop(0, 8, scale, 0)
            pltpu.sync_copy(gbuf, acc.at[colb.at[k]], add=True)
            return 0
        lax.fori_loop(0, GRP, chunk, 0)
        return 0
    lax.fori_loop(0, NCH // GRP, group, 0)
    plsc.subcore_barrier()
    pltpu.sync_copy(acc.at[pl.ds(s * RPT, RPT), :],
                    out.at[pl.ds(c * N_PAD + s * RPT, RPT), :])


@functools.cache
def _make_lx(mult, off):
    return pl.kernel(
        functools.partial(_lx_body, mult=mult, off=off),
        out_type=jax.ShapeDtypeStruct((2 * N_PAD, 64), jnp.float32),
        mesh=_mesh(),
        compiler_params=pltpu.CompilerParams(needs_layout_passes=False, use_tc_tiling_on_sc=False),
        scratch_types=[
            pltpu.VMEM((GRP, 128), jnp.int32),
            pltpu.VMEM((GRP, 128), jnp.int32),
            pltpu.VMEM((GRP * 128,), jnp.float32),
            pltpu.VMEM((128, 64), jnp.float32),
            pltpu.VMEM_SHARED((N_PAD, 64), jnp.float32),
            pltpu.SemaphoreType.DMA,
        ],
    )


def _lx64(v, rowp2d, colp2d, coef):
    return _make_lx(1, 0)(v, rowp2d, colp2d, coef).reshape(2, N_PAD, 64)


def _lx128(v, rowp2d, colp2d, coef):
    v2 = v.reshape(2 * N, 64)
    pa = _make_lx(2, 0)(v2, rowp2d, colp2d, coef).reshape(2, N_PAD, 64)
    pb = _make_lx(2, 1)(v2, rowp2d, colp2d, coef).reshape(2, N_PAD, 64)
    return pa, pb


def kernel(x, edge_index, edge_weight, adj_w1, adj_w2, conv1_w, conv1_b, conv2_w, conv2_b):
    ew = _edge_mlp(edge_weight, adj_w1, adj_w2)
    reps = edge_index.shape[-1] // 420
    train_ew = jnp.tile(ew, (reps, 1))

    pad = E_PAD - E
    rowp = jnp.concatenate([edge_index[0], jnp.zeros((pad,), edge_index.dtype)])
    colp = jnp.concatenate([edge_index[1], jnp.zeros((pad,), edge_index.dtype)])
    cp = jnp.concatenate([train_ew.reshape(-1), jnp.zeros((pad,), jnp.float32)])

    deg_parts = _deg_call()(rowp, cp).reshape(32, 200, 128)
    dis = _dis(deg_parts).reshape(N_PAD)
    norm, norm2 = _norm_call()(dis, rowp, colp, cp)

    row2d = rowp.reshape(E_PAD // 128, 128)
    col2d = colp.reshape(E_PAD // 128, 128)

    t0 = x
    t1 = _asm128(*_lx128(t0, row2d, col2d, norm))
    t2 = _asm128(*_lx128(t1, row2d, col2d, norm2), t0)
    t3 = _asm128(*_lx128(t2, row2d, col2d, norm2), t1)
    t4 = _asm128(*_lx128(t3, row2d, col2d, norm2), t2)
    h = _conv1_mm([t0, t1, t2, t3, t4], conv1_w, conv1_b)

    yy = _proj_mm(h, conv2_w)
    b4 = yy[4]
    b3 = _comb64(_lx64(b4, row2d, col2d, norm2), yy, 3)
    b2 = _comb64(_lx64(b3, row2d, col2d, norm2), yy, 2, b4)
    b1 = _comb64(_lx64(b2, row2d, col2d, norm2), yy, 1, b3)
    out = _comb64(_lx64(b1, row2d, col2d, norm), yy, 0, b2) + conv2_b

    return (out, ew, train_ew)
